# baseline (speedup 1.0000x reference)
import functools

import jax
import jax.numpy as jnp
from jax import lax
from jax.experimental import pallas as pl
from jax.experimental.pallas import tpu as pltpu
from jax.experimental.pallas import tpu_sc as plsc

E = 8
T = 2048
H = 1024
I = 1024
B = 256
PAD_T = 4096 + 8 * B
NB = PAD_T // B

NC, NS = 2, 16
NW = NC * NS
TPW = T // NW
_SC_MESH = plsc.VectorSubcoreMesh(core_axis_name="c", subcore_axis_name="s")


def _routing_body(l_ref, pos0_ref, pos1_ref, w0s_ref, w1s_ref,
                  eid_ref, nblk_ref):
    l = l_ref[...]
    ei = lax.broadcasted_iota(jnp.int32, (T, E), 1)
    m1 = jnp.max(l, axis=1, keepdims=True)
    a1 = jnp.min(jnp.where(l == m1, ei, E), axis=1, keepdims=True)
    l2 = jnp.where(ei == a1, -jnp.inf, l)
    m2 = jnp.max(l2, axis=1, keepdims=True)
    a2 = jnp.min(jnp.where(l2 == m2, ei, E), axis=1, keepdims=True)
    w0 = jax.nn.sigmoid(m1 - m2)

    oh1 = ei == a1
    oh2 = ei == a2
    C = oh1.astype(jnp.int32) + oh2.astype(jnp.int32)
    inc = C
    s = 1
    while s < T:
        inc = inc + jnp.concatenate(
            [jnp.zeros((s, E), jnp.int32), inc[:-s]], axis=0)
        s *= 2
    P = inc - C
    counts = lax.slice(inc, (T - 1, 0), (T, E))
    padded = ((counts + (B - 1)) // B) * B
    pinc = padded
    s = 1
    while s < E:
        pinc = pinc + jnp.concatenate(
            [jnp.zeros((1, s), jnp.int32), pinc[:, :-s]], axis=1)
        s *= 2
    poff = pinc - padded

    pos0 = jnp.sum(jnp.where(oh1, poff + P, 0), axis=1, keepdims=True)
    pos1 = jnp.sum(jnp.where(oh2, poff + P, 0), axis=1, keepdims=True)
    pos0_ref[...] = pos0.reshape(1, T)
    pos1_ref[...] = pos1.reshape(1, T)
    w0s_ref[...] = jnp.broadcast_to(w0, (T, 16))
    w1s_ref[...] = jnp.broadcast_to(1.0 - w0, (T, 16))

    gb = lax.broadcasted_iota(jnp.int32, (1, NB), 1) * B
    acc = jnp.zeros((1, NB), jnp.int32)
    for e in range(E):
        pe = lax.slice(poff, (0, e), (1, e + 1))
        acc = acc + (pe <= gb).astype(jnp.int32)
    eid_ref[...] = acc - 1
    nblk_ref[...] = jnp.sum(padded, keepdims=True)[:, :1] // B


def _routing(router_logits):
    return pl.pallas_call(
        _routing_body,
        out_shape=[
            jax.ShapeDtypeStruct((1, T), jnp.int32),
            jax.ShapeDtypeStruct((1, T), jnp.int32),
            jax.ShapeDtypeStruct((T, 16), jnp.float32),
            jax.ShapeDtypeStruct((T, 16), jnp.float32),
            jax.ShapeDtypeStruct((1, NB), jnp.int32),
            jax.ShapeDtypeStruct((1, 1), jnp.int32),
        ],
    )(router_logits)


def _ffn_body(eid_ref, nblk_ref, x_ref, w13_ref, w2_ref, y_ref):
    g = pl.program_id(0)

    @pl.when(g < nblk_ref[0])
    def _():
        x = x_ref[...]
        gu = lax.dot_general(x, w13_ref[0], (((1,), (1,)), ((), ())),
                             preferred_element_type=jnp.float32)
        gate = gu[:, :I]
        up = gu[:, I:]
        h = gate * jax.nn.sigmoid(gate) * up
        y_ref[...] = lax.dot_general(h, w2_ref[0], (((1,), (1,)), ((), ())),
                                     preferred_element_type=jnp.float32)


def _ffn(eid, nblk, x_sorted, w13, w2):
    spec = pltpu.PrefetchScalarGridSpec(
        num_scalar_prefetch=2,
        grid=(NB,),
        in_specs=[
            pl.BlockSpec((B, H), lambda g, eid, nb: (g, 0)),
            pl.BlockSpec((1, 2 * I, H), lambda g, eid, nb: (eid[g], 0, 0)),
            pl.BlockSpec((1, H, I), lambda g, eid, nb: (eid[g], 0, 0)),
        ],
        out_specs=pl.BlockSpec((B, H), lambda g, eid, nb: (g, 0)),
    )
    return pl.pallas_call(
        _ffn_body,
        grid_spec=spec,
        out_shape=jax.ShapeDtypeStruct((PAD_T, H), jnp.float32),
    )(eid, nblk, x_sorted, w13, w2)


@functools.partial(
    pl.kernel,
    mesh=_SC_MESH,
    compiler_params=pltpu.CompilerParams(needs_layout_passes=False),
    out_type=jax.ShapeDtypeStruct((PAD_T, H), jnp.float32),
    scratch_types=[
        pltpu.VMEM((TPW,), jnp.int32),
        pltpu.VMEM((TPW,), jnp.int32),
        pltpu.VMEM((TPW, H), jnp.float32),
        pltpu.SemaphoreType.DMA,
        pltpu.SemaphoreType.DMA,
        pltpu.SemaphoreType.DMA,
    ],
)
def _sc_dispatch(pos0_hbm, pos1_hbm, hidden_hbm, xs_hbm,
                 p0_v, p1_v, rows_v, s0, s1, s2):
    wid = lax.axis_index("s") * NC + lax.axis_index("c")
    tb = wid * TPW
    d0 = pltpu.async_copy(pos0_hbm.at[pl.ds(tb, TPW)], p0_v, s0)
    d1 = pltpu.async_copy(pos1_hbm.at[pl.ds(tb, TPW)], p1_v, s1)
    pltpu.sync_copy(hidden_hbm.at[pl.ds(tb, TPW)], rows_v)
    d0.wait()
    d1.wait()
    e0 = pltpu.async_copy(rows_v, xs_hbm.at[p0_v], s0)
    e1 = pltpu.async_copy(rows_v, xs_hbm.at[p1_v], s1)
    e0.wait()
    e1.wait()


_CTOK = TPW // 2


@functools.partial(
    pl.kernel,
    mesh=_SC_MESH,
    compiler_params=pltpu.CompilerParams(needs_layout_passes=False),
    out_type=jax.ShapeDtypeStruct((T, H), jnp.float32),
    scratch_types=[
        pltpu.VMEM((TPW,), jnp.int32),
        pltpu.VMEM((TPW,), jnp.int32),
        pltpu.VMEM((TPW, 16), jnp.float32),
        pltpu.VMEM((TPW, 16), jnp.float32),
        pltpu.VMEM((_CTOK, H), jnp.float32),
        pltpu.VMEM((_CTOK, H), jnp.float32),
        pltpu.SemaphoreType.DMA,
        pltpu.SemaphoreType.DMA,
    ],
)
def _sc_combine(pos0_hbm, pos1_hbm, w0s_hbm, w1s_hbm, y_hbm, out_hbm,
                p0_v, p1_v, w0_v, w1_v, buf_v, acc_v, s0, s1):
    wid = lax.axis_index("s") * NC + lax.axis_index("c")
    base = wid * TPW
    d0 = pltpu.async_copy(pos0_hbm.at[pl.ds(base, TPW)], p0_v, s0)
    d1 = pltpu.async_copy(pos1_hbm.at[pl.ds(base, TPW)], p1_v, s1)
    pltpu.sync_copy(w0s_hbm.at[pl.ds(base, TPW)], w0_v)
    pltpu.sync_copy(w1s_hbm.at[pl.ds(base, TPW)], w1_v)
    d0.wait()
    d1.wait()

    for c in range(TPW // _CTOK):
        pltpu.async_copy(y_hbm.at[p0_v.at[pl.ds(c * _CTOK, _CTOK)]],
                         buf_v, s0).wait()
        pltpu.async_copy(y_hbm.at[p1_v.at[pl.ds(c * _CTOK, _CTOK)]],
                         acc_v, s1).wait()

        def addrow(r, cc):
            rr = c * _CTOK + r
            wa = w0_v[rr, :]
            wb = w1_v[rr, :]
            for j in range(H // 16):
                sl = pl.ds(j * 16, 16)
                acc_v[r, sl] = buf_v[r, sl] * wa + acc_v[r, sl] * wb
            return cc

        lax.fori_loop(0, _CTOK, addrow, 0)
        pltpu.sync_copy(acc_v, out_hbm.at[pl.ds(base + c * _CTOK, _CTOK)])


def kernel(hidden_states, router_logits, w13_weight, w2_weight):
    _ABL = 4
    pos0, pos1, w0s, w1s, eid, nblk = _routing(router_logits)
    pos0 = pos0.reshape(T)
    pos1 = pos1.reshape(T)
    if _ABL == 1:
        return hidden_states * w0s[:, :1]

    x_sorted = _sc_dispatch(pos0, pos1, hidden_states)
    if _ABL == 2:
        return x_sorted[:T]

    y = _ffn(eid.reshape(NB), nblk.reshape(1), x_sorted,
             w13_weight, w2_weight)
    if _ABL == 3:
        return y[:T]

    return _sc_combine(pos0, pos1, w0s, w1s, y)
